# baseline (speedup 1.0000x reference)
import functools

import jax
import jax.numpy as jnp
from jax import lax
from jax.experimental import pallas as pl
from jax.experimental.pallas import tpu as pltpu
from jax.experimental.pallas import tpu_sc as plsc

_INFO = plsc.get_sparse_core_info()
_NC = _INFO.num_cores
_NS = _INFO.num_subcores
_NW = _NC * _NS
_BLK = 128


def _sc_gather_t(table, idx_t):
    J, I = idx_t.shape
    D = table.shape[1]
    FB = D // 8
    NB = I // _BLK
    n_units = J * NB
    upw = n_units // _NW
    assert upw % 2 == 0 and upw >= 4
    n_tiles = J * FB * NB
    mesh = plsc.VectorSubcoreMesh(core_axis_name="c", subcore_axis_name="s")

    @functools.partial(
        pl.kernel,
        out_type=jax.ShapeDtypeStruct((n_tiles, 8, _BLK), jnp.float32),
        mesh=mesh,
        scratch_types=[
            pltpu.VMEM((_BLK,), jnp.int32),
            pltpu.VMEM((_BLK,), jnp.int32),
            pltpu.VMEM((_BLK, D), jnp.float32),
            pltpu.VMEM((_BLK, D), jnp.float32),
            pltpu.VMEM((FB, 8, _BLK), jnp.float32),
            pltpu.VMEM((FB, 8, _BLK), jnp.float32),
            pltpu.SemaphoreType.DMA,
            pltpu.SemaphoreType.DMA,
            pltpu.SemaphoreType.DMA,
            pltpu.SemaphoreType.DMA,
            pltpu.SemaphoreType.DMA,
            pltpu.SemaphoreType.DMA,
        ],
        compiler_params=pltpu.CompilerParams(use_tc_tiling_on_sc=False, needs_layout_passes=False),
    )
    def k(table_hbm, idx_hbm, out_hbm, idx0, idx1, rows0, rows1, t0, t1,
          g0, g1, o0, o1, i0, i1):
        wid = lax.axis_index("s") * _NC + lax.axis_index("c")
        ubase = wid * upw
        idx_v = (idx0, idx1)
        rows_v = (rows0, rows1)
        tbuf = (t0, t1)
        g = (g0, g1)
        o = (o0, o1)
        i = (i0, i1)
        iota16 = lax.iota(jnp.int32, 16)
        rowsel = [iota16 + 16 * blk for blk in range(_BLK // 16)]

        def unit_jb(u):
            ug = ubase + u
            return ug // NB, ug % NB

        def idx_slice(u):
            j, ib = unit_jb(u)
            return idx_hbm.at[j, pl.ds(ib * _BLK, _BLK)]

        def wait_g(s):
            pltpu.make_async_copy(
                table_hbm.at[idx_v[s]], rows_v[s], g[s]).wait()

        def wait_i(s):
            pltpu.make_async_copy(idx_slice(0), idx_v[s], i[s]).wait()

        def wait_o(s):
            for _ in range(FB):
                pltpu.make_async_copy(
                    tbuf[s].at[0], out_hbm.at[0], o[s]).wait()

        def transpose(s):
            for fb in range(FB):
                for fi in range(8):
                    col = jnp.full((16,), 8 * fb + fi, jnp.int32)
                    for blk in range(_BLK // 16):
                        v = plsc.load_gather(rows_v[s], [rowsel[blk], col])
                        tbuf[s][fb, fi, pl.ds(16 * blk, 16)] = v

        def emit_out(u, s):
            j, ib = unit_jb(u)
            tb = j * (FB * NB) + ib
            for fb in range(FB):
                pltpu.async_copy(
                    tbuf[s].at[fb], out_hbm.at[tb + fb * NB], o[s])

        def step(u, s):
            ns = 1 - s
            wait_g(s)

            def prefetch_idx():
                pltpu.async_copy(idx_slice(u + 2), idx_v[s], i[s])
                return None

            pl.when(u + 2 < upw)(prefetch_idx)

            def next_gather():
                wait_i(ns)
                pltpu.async_copy(table_hbm.at[idx_v[ns]], rows_v[ns], g[ns])
                return None

            pl.when(u + 1 < upw)(next_gather)
            transpose(s)
            emit_out(u, s)

        pltpu.sync_copy(idx_slice(0), idx0)
        pltpu.async_copy(table_hbm.at[idx0], rows0, g0)
        pltpu.async_copy(idx_slice(1), idx1, i1)

        @pl.loop(0, upw // 2)
        def _(h):
            for b2 in (0, 1):
                u = 2 * h + b2

                def drain():
                    wait_o(b2)
                    return None

                pl.when(h >= 1)(drain)
                step(u, b2)

        wait_o(0)
        wait_o(1)

    return k(table, idx_t)


def kernel(token_ids, embedding):
    I, J = token_ids.shape
    D = embedding.shape[1]
    idx_t = token_ids.T.astype(jnp.int32)
    out = _sc_gather_t(embedding, idx_t)
    FB, NB = D // 8, I // _BLK
    y = out.reshape(J, FB, NB, 8, _BLK)
    y = y.transpose(2, 4, 0, 1, 3)
    return y.reshape(I, J, D)


# speedup vs baseline: 1.2592x; 1.2592x over previous
import functools

import jax
import jax.numpy as jnp
from jax import lax
from jax.experimental import pallas as pl
from jax.experimental.pallas import tpu as pltpu
from jax.experimental.pallas import tpu_sc as plsc

_INFO = plsc.get_sparse_core_info()
_NC = _INFO.num_cores
_NS = _INFO.num_subcores
_NW = _NC * _NS
_BLK = 128


def _sc_gather_t(table, idx_t):
    J, I = idx_t.shape
    D = table.shape[1]
    FB = D // 8
    NB = I // _BLK
    n_units = J * NB
    upw = n_units // _NW
    assert upw % 2 == 0 and upw >= 4
    n_tiles = J * FB * NB
    mesh = plsc.VectorSubcoreMesh(core_axis_name="c", subcore_axis_name="s")

    @functools.partial(
        pl.kernel,
        out_type=jax.ShapeDtypeStruct((n_tiles, 8, _BLK), jnp.float32),
        mesh=mesh,
        scratch_types=[
            pltpu.VMEM((_BLK,), jnp.int32),
            pltpu.VMEM((_BLK,), jnp.int32),
            pltpu.VMEM((_BLK, D), jnp.float32),
            pltpu.VMEM((_BLK, D), jnp.float32),
            pltpu.VMEM((FB, 8, _BLK), jnp.float32),
            pltpu.VMEM((FB, 8, _BLK), jnp.float32),
            pltpu.SemaphoreType.DMA,
            pltpu.SemaphoreType.DMA,
            pltpu.SemaphoreType.DMA,
            pltpu.SemaphoreType.DMA,
            pltpu.SemaphoreType.DMA,
            pltpu.SemaphoreType.DMA,
        ],
        compiler_params=pltpu.CompilerParams(use_tc_tiling_on_sc=False, needs_layout_passes=False),
    )
    def k(table_hbm, idx_hbm, out_hbm, idx0, idx1, rows0, rows1, t0, t1,
          g0, g1, o0, o1, i0, i1):
        wid = lax.axis_index("s") * _NC + lax.axis_index("c")
        ubase = wid * upw
        idx_v = (idx0, idx1)
        rows_v = (rows0, rows1)
        tbuf = (t0, t1)
        g = (g0, g1)
        o = (o0, o1)
        i = (i0, i1)
        iota16 = lax.iota(jnp.int32, 16)
        rowsel = [iota16 + 16 * blk for blk in range(_BLK // 16)]

        def unit_jb(u):
            ug = ubase + u
            return ug // NB, ug % NB

        def idx_slice(u):
            j, ib = unit_jb(u)
            return idx_hbm.at[j, pl.ds(ib * _BLK, _BLK)]

        def wait_g(s):
            pltpu.make_async_copy(
                table_hbm.at[idx_v[s]], rows_v[s], g[s]).wait()

        def wait_i(s):
            pltpu.make_async_copy(idx_slice(0), idx_v[s], i[s]).wait()

        def wait_o(s):
            for _ in range(FB):
                pltpu.make_async_copy(
                    tbuf[s].at[0], out_hbm.at[0], o[s]).wait()

        def transpose(s):
            nblk = _BLK // 16
            for fb in range(FB):
                for fi in range(8):
                    col = jnp.full((16,), 8 * fb + fi, jnp.int32)
                    vs = [plsc.load_gather(rows_v[s], [rowsel[blk], col])
                          for blk in range(nblk)]
                    for blk in range(nblk):
                        tbuf[s][fb, fi, pl.ds(16 * blk, 16)] = vs[blk]

        def emit_out(u, s):
            j, ib = unit_jb(u)
            tb = j * (FB * NB) + ib
            for fb in range(FB):
                pltpu.async_copy(
                    tbuf[s].at[fb], out_hbm.at[tb + fb * NB], o[s])

        def step(u, s):
            ns = 1 - s
            wait_g(s)

            def prefetch_idx():
                pltpu.async_copy(idx_slice(u + 2), idx_v[s], i[s])
                return None

            pl.when(u + 2 < upw)(prefetch_idx)

            def next_gather():
                wait_i(ns)
                pltpu.async_copy(table_hbm.at[idx_v[ns]], rows_v[ns], g[ns])
                return None

            pl.when(u + 1 < upw)(next_gather)
            transpose(s)
            emit_out(u, s)

        pltpu.sync_copy(idx_slice(0), idx0)
        pltpu.async_copy(table_hbm.at[idx0], rows0, g0)
        pltpu.async_copy(idx_slice(1), idx1, i1)

        @pl.loop(0, upw // 2)
        def _(h):
            for b2 in (0, 1):
                u = 2 * h + b2

                def drain():
                    wait_o(b2)
                    return None

                pl.when(h >= 1)(drain)
                step(u, b2)

        wait_o(0)
        wait_o(1)

    return k(table, idx_t)


def kernel(token_ids, embedding):
    I, J = token_ids.shape
    D = embedding.shape[1]
    idx_t = token_ids.T.astype(jnp.int32)
    out = _sc_gather_t(embedding, idx_t)
    FB, NB = D // 8, I // _BLK
    y = out.reshape(J, FB, NB, 8, _BLK)
    y = y.transpose(2, 4, 0, 1, 3)
    return y.reshape(I, J, D)
